# baseline (speedup 1.0000x reference)
import functools

import jax
import jax.numpy as jnp
from jax import lax
from jax.experimental import pallas as pl
from jax.experimental.pallas import tpu as pltpu
from jax.experimental.pallas import tpu_sc as plsc

S = 2048
D = 768
E = 8
DFF = 3072
TM = 256
PAD = S * 2 + E * TM
NT = PAD // TM

NW = 32
PAIRS = 2 * S
BP = PAIRS // NW
BT = S // NW
WL = 128


def _cumsum_excl(a):
    incl = a
    k = 1
    n = a.shape[0]
    while k < n:
        zero = jnp.zeros((k, a.shape[1]), a.dtype)
        incl = incl + jnp.concatenate([zero, incl[:-k]], axis=0)
        k *= 2
    return incl - a


def _router_meta_kernel(
    x_ref, wr_ref, br_ref, d0_ref, d1_ref, w0_ref, w1_ref, te_ref, runid_ref,
    runex_ref,
):
    x = x_ref[...]
    logits = jnp.dot(x, wr_ref[...], preferred_element_type=jnp.float32)
    logits = logits + br_ref[...]
    ecols = lax.broadcasted_iota(jnp.int32, (S, E), 1)

    m0 = jnp.max(logits, axis=1, keepdims=True)
    a0 = jnp.min(jnp.where(logits == m0, ecols, E), axis=1, keepdims=True)
    rest = jnp.where(ecols == a0, -jnp.inf, logits)
    m1 = jnp.max(rest, axis=1, keepdims=True)
    a1 = jnp.min(jnp.where(rest == m1, ecols, E), axis=1, keepdims=True)

    e1 = jnp.exp(m1 - m0)
    w0 = 1.0 / (1.0 + e1)
    w1 = e1 / (1.0 + e1)

    oh0 = (ecols == a0).astype(jnp.float32)
    oh1 = (ecols == a1).astype(jnp.float32)
    c0x = _cumsum_excl(oh0)
    c1x = _cumsum_excl(oh1)
    cnt0 = jnp.sum(oh0, axis=0, keepdims=True)
    counts = cnt0 + jnp.sum(oh1, axis=0, keepdims=True)

    padded = jnp.ceil(counts / TM) * TM
    r8 = lax.broadcasted_iota(jnp.int32, (E, E), 0)
    c8 = lax.broadcasted_iota(jnp.int32, (E, E), 1)
    tri_incl = (r8 <= c8).astype(jnp.float32)
    incl = jnp.dot(padded, tri_incl, preferred_element_type=jnp.float32)
    offs = incl - padded

    r0 = jnp.sum(c0x * oh0, axis=1, keepdims=True)
    r1 = jnp.sum((cnt0 + c1x) * oh1, axis=1, keepdims=True)
    d0 = jnp.sum(offs * oh0, axis=1, keepdims=True) + r0
    d1 = jnp.sum(offs * oh1, axis=1, keepdims=True) + r1
    d0_ref[...] = d0.astype(jnp.int32)
    d1_ref[...] = d1.astype(jnp.int32)
    w0_ref[...] = jnp.broadcast_to(w0, (S, WL))
    w1_ref[...] = jnp.broadcast_to(w1, (S, WL))

    rs = lax.broadcasted_iota(jnp.int32, (NT, 1), 0).astype(jnp.float32) * TM
    te = jnp.sum((incl <= rs).astype(jnp.int32), axis=1, keepdims=True)
    te_ref[...] = jnp.minimum(te, E - 1)

    id8 = (r8 == c8).astype(jnp.float32)
    incl_col = jnp.sum(incl * id8, axis=1, keepdims=True)
    rs_row = lax.broadcasted_iota(jnp.int32, (1, NT), 1).astype(jnp.float32) * TM
    te_row = jnp.sum((incl_col <= rs_row).astype(jnp.int32), axis=0, keepdims=True)
    te_row = jnp.minimum(te_row, E - 1).astype(jnp.float32)
    te_prev = jnp.concatenate([te_row[:, :1], te_row[:, :-1]], axis=1)
    lane = lax.broadcasted_iota(jnp.int32, (1, NT), 1)
    chg = (te_row != te_prev).astype(jnp.float32)
    first = jnp.maximum(chg, (lane == 0).astype(jnp.float32))
    rN = lax.broadcasted_iota(jnp.int32, (NT, NT), 0)
    cN = lax.broadcasted_iota(jnp.int32, (NT, NT), 1)
    runid_col = jnp.sum(chg * (cN <= rN).astype(jnp.float32), axis=1, keepdims=True)
    runid_ref[...] = runid_col.astype(jnp.int32)
    idN = (rN == cN).astype(jnp.float32)
    runid_lane = jnp.sum(runid_col * idN, axis=0, keepdims=True)
    r_col = lax.broadcasted_iota(jnp.int32, (NT, 1), 0).astype(jnp.float32)
    sel = (runid_lane == r_col).astype(jnp.float32)
    runex = jnp.sum(sel * (te_row * first), axis=1, keepdims=True)
    runex_ref[...] = runex.astype(jnp.int32)


def _dispatch_body(x_ref, dcat_ref, wcat_ref, xg_ref, wg_ref,
                   d0_v, d1_v, rows_v, w0_v, w1_v, sem):
    wid = lax.axis_index("s") * 2 + lax.axis_index("c")
    tbase = wid * BT
    pltpu.sync_copy(dcat_ref.at[pl.ds(tbase, BT)], d0_v)
    pltpu.sync_copy(dcat_ref.at[pl.ds(S + tbase, BT)], d1_v)
    pltpu.sync_copy(x_ref.at[pl.ds(tbase, BT)], rows_v)
    pltpu.sync_copy(wcat_ref.at[pl.ds(tbase, BT)], w0_v)
    pltpu.sync_copy(wcat_ref.at[pl.ds(S + tbase, BT)], w1_v)
    cp0 = pltpu.async_copy(rows_v, xg_ref.at[d0_v], sem)
    cp1 = pltpu.async_copy(rows_v, xg_ref.at[d1_v], sem)
    cp2 = pltpu.async_copy(w0_v, wg_ref.at[d0_v], sem)
    cp3 = pltpu.async_copy(w1_v, wg_ref.at[d1_v], sem)
    cp0.wait()
    cp1.wait()
    cp2.wait()
    cp3.wait()


def _combine_body(y_ref, dcat_ref, out_ref, idx_v, r0_v, r1_v, sem):
    wid = lax.axis_index("s") * 2 + lax.axis_index("c")
    tbase = wid * BT
    pltpu.sync_copy(dcat_ref.at[pl.ds(tbase, BT)], idx_v)
    pltpu.async_copy(y_ref.at[idx_v], r0_v, sem).wait()
    pltpu.sync_copy(dcat_ref.at[pl.ds(S + tbase, BT)], idx_v)
    pltpu.async_copy(y_ref.at[idx_v], r1_v, sem).wait()

    def body(i, carry):
        for j in range(D // 16):
            sl = pl.ds(j * 16, 16)
            r0_v[i, sl] = r0_v[i, sl] + r1_v[i, sl]
        return carry

    lax.fori_loop(0, BT, body, 0)
    pltpu.sync_copy(r0_v, out_ref.at[pl.ds(tbase, BT)])


def _ffn_kernel(
    te_ref, runid_ref, runex_ref,
    xg_ref, w1_hbm, b1_ref, w2_hbm, b2_ref, wg_ref, y_ref,
    w1buf, w2buf, sem1, sem2,
):
    del te_ref
    t = pl.program_id(0)
    rid = runid_ref[t]
    slot = lax.rem(rid, 2)
    max_rid = runid_ref[NT - 1]

    def issue(r, slot_):
        e = runex_ref[r]
        pltpu.make_async_copy(
            w1_hbm.at[pl.ds(e, 1)], w1buf.at[pl.ds(slot_, 1)], sem1.at[slot_]
        ).start()
        pltpu.make_async_copy(
            w2_hbm.at[pl.ds(e, 1)], w2buf.at[pl.ds(slot_, 1)], sem2.at[slot_]
        ).start()

    is_first = jnp.logical_or(t == 0, rid != runid_ref[jnp.maximum(t - 1, 0)])

    @pl.when(t == 0)
    def _():
        issue(0, 0)

    @pl.when(jnp.logical_and(t == 0, max_rid >= 1))
    def _():
        issue(1, 1)

    @pl.when(jnp.logical_and(is_first, jnp.logical_and(t > 0, rid < max_rid)))
    def _():
        issue(rid + 1, lax.rem(rid + 1, 2))

    @pl.when(is_first)
    def _():
        pltpu.make_async_copy(
            w1_hbm.at[pl.ds(0, 1)], w1buf.at[pl.ds(slot, 1)], sem1.at[slot]
        ).wait()
        pltpu.make_async_copy(
            w2_hbm.at[pl.ds(0, 1)], w2buf.at[pl.ds(slot, 1)], sem2.at[slot]
        ).wait()

    xgi = xg_ref[...]
    xa = lax.bitcast_convert_type(xgi << 16, jnp.float32)
    xb = lax.bitcast_convert_type(
        jnp.bitwise_and(xgi, jnp.int32(-65536)), jnp.float32
    )
    xg = jnp.concatenate([xa, xb], axis=1)
    w1 = w1buf[pl.ds(slot, 1)][0]
    w2 = w2buf[pl.ds(slot, 1)][0]
    h = jnp.dot(xg, w1, preferred_element_type=jnp.float32) + b1_ref[0]
    h = 0.5 * h * (1.0 + lax.erf(h * 0.7071067811865476))
    y = jnp.dot(h, w2, preferred_element_type=jnp.float32) + b2_ref[0]
    y_ref[...] = y * wg_ref[:, :1]


def kernel(x, Wr, br, W1, b1, W2, b2):
    Bs, Ss, Dd = x.shape
    x2 = x.reshape(S, D)

    d0, d1, w0b, w1b, te, runid, runex = pl.pallas_call(
        _router_meta_kernel,
        out_shape=[
            jax.ShapeDtypeStruct((S, 1), jnp.int32),
            jax.ShapeDtypeStruct((S, 1), jnp.int32),
            jax.ShapeDtypeStruct((S, WL), jnp.float32),
            jax.ShapeDtypeStruct((S, WL), jnp.float32),
            jax.ShapeDtypeStruct((NT, 1), jnp.int32),
            jax.ShapeDtypeStruct((NT, 1), jnp.int32),
            jax.ShapeDtypeStruct((NT, 1), jnp.int32),
        ],
    )(x2, Wr, br.reshape(1, E))

    dcat = jnp.concatenate([d0.reshape(S), d1.reshape(S)])
    wcat = jnp.concatenate([w0b, w1b])
    te_flat = te.reshape(NT)
    runid_flat = runid.reshape(NT)
    runex_flat = runex.reshape(NT)

    mesh = plsc.VectorSubcoreMesh(core_axis_name="c", subcore_axis_name="s")

    dispatch = functools.partial(
        pl.kernel,
        mesh=mesh,
        out_type=[
            jax.ShapeDtypeStruct((PAD, D // 2), jnp.int32),
            jax.ShapeDtypeStruct((PAD, WL), jnp.float32),
        ],
        scratch_types=[
            pltpu.VMEM((BT,), jnp.int32),
            pltpu.VMEM((BT,), jnp.int32),
            pltpu.VMEM((BT, D // 2), jnp.int32),
            pltpu.VMEM((BT, WL), jnp.float32),
            pltpu.VMEM((BT, WL), jnp.float32),
            pltpu.SemaphoreType.DMA,
        ],
    )(_dispatch_body)
    xh = jnp.stack([x2[:, : D // 2], x2[:, D // 2 :]], axis=-1)
    x2p = lax.bitcast_convert_type(
        xh.astype(jnp.bfloat16), jnp.int32
    )
    xg, wg = dispatch(x2p, dcat, wcat)

    grid_spec = pltpu.PrefetchScalarGridSpec(
        num_scalar_prefetch=3,
        grid=(NT,),
        in_specs=[
            pl.BlockSpec((TM, D // 2), lambda t, te, ri, rx: (t, 0)),
            pl.BlockSpec(memory_space=pltpu.MemorySpace.HBM),
            pl.BlockSpec((1, 1, DFF), lambda t, te, ri, rx: (te[t], 0, 0)),
            pl.BlockSpec(memory_space=pltpu.MemorySpace.HBM),
            pl.BlockSpec((1, 1, D), lambda t, te, ri, rx: (te[t], 0, 0)),
            pl.BlockSpec((TM, WL), lambda t, te, ri, rx: (t, 0)),
        ],
        out_specs=pl.BlockSpec((TM, D), lambda t, te, ri, rx: (t, 0)),
        scratch_shapes=[
            pltpu.VMEM((2, D, DFF), jnp.float32),
            pltpu.VMEM((2, DFF, D), jnp.float32),
            pltpu.SemaphoreType.DMA((2,)),
            pltpu.SemaphoreType.DMA((2,)),
        ],
    )
    y = pl.pallas_call(
        _ffn_kernel,
        grid_spec=grid_spec,
        out_shape=jax.ShapeDtypeStruct((PAD, D), jnp.float32),
    )(
        te_flat, runid_flat, runex_flat,
        xg, W1, b1.reshape(E, 1, DFF), W2, b2.reshape(E, 1, D), wg,
    )

    combine = functools.partial(
        pl.kernel,
        mesh=mesh,
        out_type=jax.ShapeDtypeStruct((S, D), jnp.float32),
        scratch_types=[
            pltpu.VMEM((BT,), jnp.int32),
            pltpu.VMEM((BT, D), jnp.float32),
            pltpu.VMEM((BT, D), jnp.float32),
            pltpu.SemaphoreType.DMA,
        ],
    )(_combine_body)
    out = combine(y, dcat)

    return out.reshape(Bs, Ss, Dd)


# speedup vs baseline: 1.0374x; 1.0374x over previous
import functools

import jax
import jax.numpy as jnp
from jax import lax
from jax.experimental import pallas as pl
from jax.experimental.pallas import tpu as pltpu
from jax.experimental.pallas import tpu_sc as plsc

S = 2048
D = 768
E = 8
DFF = 3072
TM = 256
PAD = S * 2 + E * TM
NT = PAD // TM

NW = 32
PAIRS = 2 * S
BP = PAIRS // NW
BT = S // NW
WL = 128


def _cumsum_excl(a):
    incl = a
    k = 1
    n = a.shape[0]
    while k < n:
        zero = jnp.zeros((k, a.shape[1]), a.dtype)
        incl = incl + jnp.concatenate([zero, incl[:-k]], axis=0)
        k *= 2
    return incl - a


def _router_meta_kernel(
    x_ref, wr_ref, br_ref, dcat_ref, wcat_ref, te_ref, runid_ref, runex_ref
):
    x = x_ref[...]
    logits = jnp.dot(x, wr_ref[...], preferred_element_type=jnp.float32)
    logits = logits + br_ref[...]
    ecols = lax.broadcasted_iota(jnp.int32, (S, E), 1)

    m0 = jnp.max(logits, axis=1, keepdims=True)
    a0 = jnp.min(jnp.where(logits == m0, ecols, E), axis=1, keepdims=True)
    rest = jnp.where(ecols == a0, -jnp.inf, logits)
    m1 = jnp.max(rest, axis=1, keepdims=True)
    a1 = jnp.min(jnp.where(rest == m1, ecols, E), axis=1, keepdims=True)

    e1 = jnp.exp(m1 - m0)
    w0 = 1.0 / (1.0 + e1)
    w1 = e1 / (1.0 + e1)

    oh0 = (ecols == a0).astype(jnp.float32)
    oh1 = (ecols == a1).astype(jnp.float32)
    c0x = _cumsum_excl(oh0)
    c1x = _cumsum_excl(oh1)
    cnt0 = jnp.sum(oh0, axis=0, keepdims=True)
    counts = cnt0 + jnp.sum(oh1, axis=0, keepdims=True)

    padded = jnp.ceil(counts / TM) * TM
    r8 = lax.broadcasted_iota(jnp.int32, (E, E), 0)
    c8 = lax.broadcasted_iota(jnp.int32, (E, E), 1)
    tri_incl = (r8 <= c8).astype(jnp.float32)
    incl = jnp.dot(padded, tri_incl, preferred_element_type=jnp.float32)
    offs = incl - padded

    r0 = jnp.sum(c0x * oh0, axis=1, keepdims=True)
    r1 = jnp.sum((cnt0 + c1x) * oh1, axis=1, keepdims=True)
    d0 = jnp.sum(offs * oh0, axis=1, keepdims=True) + r0
    d1 = jnp.sum(offs * oh1, axis=1, keepdims=True) + r1
    dcat_ref[:S] = d0.astype(jnp.int32)
    dcat_ref[S:] = d1.astype(jnp.int32)
    wcat_ref[:S] = jnp.broadcast_to(w0, (S, WL))
    wcat_ref[S:] = jnp.broadcast_to(w1, (S, WL))

    rs = lax.broadcasted_iota(jnp.int32, (NT, 1), 0).astype(jnp.float32) * TM
    te = jnp.sum((incl <= rs).astype(jnp.int32), axis=1, keepdims=True)
    te_ref[...] = jnp.minimum(te, E - 1)

    id8 = (r8 == c8).astype(jnp.float32)
    incl_col = jnp.sum(incl * id8, axis=1, keepdims=True)
    rs_row = lax.broadcasted_iota(jnp.int32, (1, NT), 1).astype(jnp.float32) * TM
    te_row = jnp.sum((incl_col <= rs_row).astype(jnp.int32), axis=0, keepdims=True)
    te_row = jnp.minimum(te_row, E - 1).astype(jnp.float32)
    te_prev = jnp.concatenate([te_row[:, :1], te_row[:, :-1]], axis=1)
    lane = lax.broadcasted_iota(jnp.int32, (1, NT), 1)
    chg = (te_row != te_prev).astype(jnp.float32)
    first = jnp.maximum(chg, (lane == 0).astype(jnp.float32))
    rN = lax.broadcasted_iota(jnp.int32, (NT, NT), 0)
    cN = lax.broadcasted_iota(jnp.int32, (NT, NT), 1)
    runid_col = jnp.sum(chg * (cN <= rN).astype(jnp.float32), axis=1, keepdims=True)
    runid_ref[...] = runid_col.astype(jnp.int32)
    idN = (rN == cN).astype(jnp.float32)
    runid_lane = jnp.sum(runid_col * idN, axis=0, keepdims=True)
    r_col = lax.broadcasted_iota(jnp.int32, (NT, 1), 0).astype(jnp.float32)
    sel = (runid_lane == r_col).astype(jnp.float32)
    runex = jnp.sum(sel * (te_row * first), axis=1, keepdims=True)
    runex_ref[...] = runex.astype(jnp.int32)


def _dispatch_body(x_ref, dcat_ref, wcat_ref, xg_ref, wg_ref,
                   d0_v, d1_v, rows_v, w0_v, w1_v, sem):
    wid = lax.axis_index("s") * 2 + lax.axis_index("c")
    tbase = wid * BT
    pltpu.sync_copy(dcat_ref.at[pl.ds(tbase, BT)], d0_v)
    pltpu.sync_copy(dcat_ref.at[pl.ds(S + tbase, BT)], d1_v)
    pltpu.sync_copy(x_ref.at[pl.ds(tbase, BT)], rows_v)
    pltpu.sync_copy(wcat_ref.at[pl.ds(tbase, BT)], w0_v)
    pltpu.sync_copy(wcat_ref.at[pl.ds(S + tbase, BT)], w1_v)
    cp0 = pltpu.async_copy(rows_v, xg_ref.at[d0_v], sem)
    cp1 = pltpu.async_copy(rows_v, xg_ref.at[d1_v], sem)
    cp2 = pltpu.async_copy(w0_v, wg_ref.at[d0_v], sem)
    cp3 = pltpu.async_copy(w1_v, wg_ref.at[d1_v], sem)
    cp0.wait()
    cp1.wait()
    cp2.wait()
    cp3.wait()


def _combine_body(y_ref, dcat_ref, out_ref, idx_v, r0_v, r1_v, sem):
    wid = lax.axis_index("s") * 2 + lax.axis_index("c")
    tbase = wid * BT
    pltpu.sync_copy(dcat_ref.at[pl.ds(tbase, BT)], idx_v)
    pltpu.async_copy(y_ref.at[idx_v], r0_v, sem).wait()
    pltpu.sync_copy(dcat_ref.at[pl.ds(S + tbase, BT)], idx_v)
    pltpu.async_copy(y_ref.at[idx_v], r1_v, sem).wait()

    def body(i, carry):
        for j in range(D // 16):
            sl = pl.ds(j * 16, 16)
            r0_v[i, sl] = r0_v[i, sl] + r1_v[i, sl]
        return carry

    lax.fori_loop(0, BT, body, 0)
    pltpu.sync_copy(r0_v, out_ref.at[pl.ds(tbase, BT)])


def _ffn_kernel(
    te_ref, runid_ref, runex_ref,
    xg_ref, w1_hbm, b1_ref, w2_hbm, b2_ref, wg_ref, y_ref,
    w1buf, w2buf, sem1, sem2,
):
    del te_ref
    t = pl.program_id(0)
    rid = runid_ref[t]
    slot = lax.rem(rid, 2)
    max_rid = runid_ref[NT - 1]

    def issue(r, slot_):
        e = runex_ref[r]
        pltpu.make_async_copy(
            w1_hbm.at[pl.ds(e, 1)], w1buf.at[pl.ds(slot_, 1)], sem1.at[slot_]
        ).start()
        pltpu.make_async_copy(
            w2_hbm.at[pl.ds(e, 1)], w2buf.at[pl.ds(slot_, 1)], sem2.at[slot_]
        ).start()

    is_first = jnp.logical_or(t == 0, rid != runid_ref[jnp.maximum(t - 1, 0)])

    @pl.when(t == 0)
    def _():
        issue(0, 0)

    @pl.when(jnp.logical_and(t == 0, max_rid >= 1))
    def _():
        issue(1, 1)

    @pl.when(jnp.logical_and(is_first, jnp.logical_and(t > 0, rid < max_rid)))
    def _():
        issue(rid + 1, lax.rem(rid + 1, 2))

    @pl.when(is_first)
    def _():
        pltpu.make_async_copy(
            w1_hbm.at[pl.ds(0, 1)], w1buf.at[pl.ds(slot, 1)], sem1.at[slot]
        ).wait()
        pltpu.make_async_copy(
            w2_hbm.at[pl.ds(0, 1)], w2buf.at[pl.ds(slot, 1)], sem2.at[slot]
        ).wait()

    xg = xg_ref[...]
    w1 = w1buf[pl.ds(slot, 1)][0]
    w2 = w2buf[pl.ds(slot, 1)][0]
    h = jnp.dot(xg, w1, preferred_element_type=jnp.float32) + b1_ref[0]
    h = 0.5 * h * (1.0 + lax.erf(h * 0.7071067811865476))
    y = jnp.dot(h, w2, preferred_element_type=jnp.float32) + b2_ref[0]
    y_ref[...] = y * wg_ref[:, :1]


def kernel(x, Wr, br, W1, b1, W2, b2):
    Bs, Ss, Dd = x.shape
    x2 = x.reshape(S, D)

    dcat2, wcat, te, runid, runex = pl.pallas_call(
        _router_meta_kernel,
        out_shape=[
            jax.ShapeDtypeStruct((PAIRS, 1), jnp.int32),
            jax.ShapeDtypeStruct((PAIRS, WL), jnp.float32),
            jax.ShapeDtypeStruct((NT, 1), jnp.int32),
            jax.ShapeDtypeStruct((NT, 1), jnp.int32),
            jax.ShapeDtypeStruct((NT, 1), jnp.int32),
        ],
    )(x2, Wr, br.reshape(1, E))

    dcat = dcat2.reshape(PAIRS)
    te_flat = te.reshape(NT)
    runid_flat = runid.reshape(NT)
    runex_flat = runex.reshape(NT)

    mesh = plsc.VectorSubcoreMesh(core_axis_name="c", subcore_axis_name="s")

    dispatch = functools.partial(
        pl.kernel,
        mesh=mesh,
        out_type=[
            jax.ShapeDtypeStruct((PAD, D), jnp.float32),
            jax.ShapeDtypeStruct((PAD, WL), jnp.float32),
        ],
        scratch_types=[
            pltpu.VMEM((BT,), jnp.int32),
            pltpu.VMEM((BT,), jnp.int32),
            pltpu.VMEM((BT, D), jnp.float32),
            pltpu.VMEM((BT, WL), jnp.float32),
            pltpu.VMEM((BT, WL), jnp.float32),
            pltpu.SemaphoreType.DMA,
        ],
    )(_dispatch_body)
    xg, wg = dispatch(x2, dcat, wcat)

    grid_spec = pltpu.PrefetchScalarGridSpec(
        num_scalar_prefetch=3,
        grid=(NT,),
        in_specs=[
            pl.BlockSpec((TM, D), lambda t, te, ri, rx: (t, 0)),
            pl.BlockSpec(memory_space=pltpu.MemorySpace.HBM),
            pl.BlockSpec((1, 1, DFF), lambda t, te, ri, rx: (te[t], 0, 0)),
            pl.BlockSpec(memory_space=pltpu.MemorySpace.HBM),
            pl.BlockSpec((1, 1, D), lambda t, te, ri, rx: (te[t], 0, 0)),
            pl.BlockSpec((TM, WL), lambda t, te, ri, rx: (t, 0)),
        ],
        out_specs=pl.BlockSpec((TM, D), lambda t, te, ri, rx: (t, 0)),
        scratch_shapes=[
            pltpu.VMEM((2, D, DFF), jnp.float32),
            pltpu.VMEM((2, DFF, D), jnp.float32),
            pltpu.SemaphoreType.DMA((2,)),
            pltpu.SemaphoreType.DMA((2,)),
        ],
    )
    y = pl.pallas_call(
        _ffn_kernel,
        grid_spec=grid_spec,
        out_shape=jax.ShapeDtypeStruct((PAD, D), jnp.float32),
    )(
        te_flat, runid_flat, runex_flat,
        xg, W1, b1.reshape(E, 1, DFF), W2, b2.reshape(E, 1, D), wg,
    )

    combine = functools.partial(
        pl.kernel,
        mesh=mesh,
        out_type=jax.ShapeDtypeStruct((S, D), jnp.float32),
        scratch_types=[
            pltpu.VMEM((BT,), jnp.int32),
            pltpu.VMEM((BT, D), jnp.float32),
            pltpu.VMEM((BT, D), jnp.float32),
            pltpu.SemaphoreType.DMA,
        ],
    )(_combine_body)
    out = combine(y, dcat)

    return out.reshape(Bs, Ss, Dd)
